# baseline (speedup 1.0000x reference)
import functools

import jax
import jax.numpy as jnp
from jax import lax
from jax.experimental import pallas as pl
from jax.experimental.pallas import tpu as pltpu
from jax.experimental.pallas import tpu_sc as plsc

VOCAB = 1000000
EMBED_DIM = 64
LORA_R = 16
LORA_SCALING = 2.0
LANES = 16
CHUNK = 128
NBUF = 2


@functools.lru_cache(maxsize=None)
def _make_kernel(n_tokens: int):
    info = plsc.get_sparse_core_info()
    num_cores, num_subcores = info.num_cores, info.num_subcores
    num_workers = num_cores * num_subcores
    per_worker = n_tokens // num_workers
    assert per_worker * num_workers == n_tokens
    assert per_worker % CHUNK == 0
    n_chunks = per_worker // CHUNK
    d_groups = EMBED_DIM // LANES
    groups = CHUNK // LANES

    mesh = plsc.VectorSubcoreMesh(core_axis_name="c", subcore_axis_name="s")

    @functools.partial(
        pl.kernel,
        mesh=mesh,
        compiler_params=pltpu.CompilerParams(use_tc_tiling_on_sc=False),
        out_type=jax.ShapeDtypeStruct((n_tokens, EMBED_DIM), jnp.float32),
        scratch_types=[
            pltpu.VMEM((NBUF, CHUNK), jnp.int32),
            pltpu.VMEM((NBUF, CHUNK), jnp.int32),
            pltpu.VMEM((NBUF, LORA_R, CHUNK), jnp.int32),
            pltpu.VMEM((NBUF, CHUNK, 2 * EMBED_DIM), jnp.float32),
            pltpu.VMEM((NBUF, LORA_R, CHUNK), jnp.float32),
            pltpu.VMEM((NBUF, CHUNK, EMBED_DIM), jnp.float32),
            pltpu.VMEM((LORA_R, EMBED_DIM), jnp.float32),
            [pltpu.SemaphoreType.DMA] * NBUF,
            [pltpu.SemaphoreType.DMA] * NBUF,
            [pltpu.SemaphoreType.DMA] * NBUF,
        ],
    )
    def sc_kernel(tbl_hbm, a_hbm, bt_hbm, idx_hbm, out_hbm,
                  idx_v, sup_v, idx2_v, rows_v, a_vt, out_v, b_v,
                  sem_rows, sem_a, sem_out):
        wid = lax.axis_index("s") * num_cores + lax.axis_index("c")
        base0 = wid * per_worker

        pltpu.sync_copy(bt_hbm, b_v)

        def fetch(c, buf):
            base = base0 + c * CHUNK
            pltpu.sync_copy(idx_hbm.at[pl.ds(base, CHUNK)], idx_v.at[buf])
            for g in range(groups):
                iv = idx_v[buf, pl.ds(g * LANES, LANES)]
                sup_v[buf, pl.ds(g * LANES, LANES)] = iv >> 1
                for j in range(LORA_R):
                    idx2_v[buf, j, pl.ds(g * LANES, LANES)] = iv + j * VOCAB
            pltpu.async_copy(tbl_hbm.at[sup_v.at[buf]], rows_v.at[buf],
                             sem_rows[buf])
            for j in range(LORA_R):
                pltpu.async_copy(a_hbm.at[idx2_v.at[buf, j]],
                                 a_vt.at[buf, j], sem_a[buf])

        def drain_gathers(buf):
            pltpu.make_async_copy(tbl_hbm.at[sup_v.at[buf]], rows_v.at[buf],
                                  sem_rows[buf]).wait()
            for j in range(LORA_R):
                pltpu.make_async_copy(a_hbm.at[idx2_v.at[buf, j]],
                                      a_vt.at[buf, j], sem_a[buf]).wait()

        def compute(buf):
            def group_body(g, carry):
                t0 = g * LANES
                avs = [a_vt[buf, j, pl.ds(t0, LANES)] for j in range(LORA_R)]
                offv = (idx_v[buf, pl.ds(t0, LANES)] & 1) * EMBED_DIM
                for half in range(2):
                    js = range(half * 8, half * 8 + 8)
                    bh = {j: [b_v[j, pl.ds(dg * LANES, LANES)]
                              for dg in range(d_groups)] for j in js}
                    for l in range(LANES):
                        t = t0 + l
                        if half == 0:
                            off = offv[l]
                            accs = [rows_v[buf, t,
                                           pl.ds(off + dg * LANES, LANES)]
                                    for dg in range(d_groups)]
                        else:
                            accs = [out_v[buf, t, pl.ds(dg * LANES, LANES)]
                                    for dg in range(d_groups)]
                        for j in js:
                            s = avs[j][l]
                            for dg in range(d_groups):
                                accs[dg] = accs[dg] + s * bh[j][dg]
                        for dg in range(d_groups):
                            out_v[buf, t, pl.ds(dg * LANES, LANES)] = accs[dg]
                return carry

            lax.fori_loop(0, groups, group_body, 0)

        def issue_out(c, buf):
            base = base0 + c * CHUNK
            pltpu.async_copy(out_v.at[buf], out_hbm.at[pl.ds(base, CHUNK)],
                             sem_out[buf])

        def drain_out(c, buf):
            base = base0 + c * CHUNK
            pltpu.make_async_copy(out_v.at[buf],
                                  out_hbm.at[pl.ds(base, CHUNK)],
                                  sem_out[buf]).wait()

        fetch(0, 0)

        assert n_chunks % NBUF == 0

        def pair_body(c2, carry):
            for buf in range(NBUF):
                c = c2 * NBUF + buf
                nb = (buf + 1) % NBUF

                @pl.when(c + 1 < n_chunks)
                def _(c=c, nb=nb):
                    fetch(c + 1, nb)

                drain_gathers(buf)

                @pl.when(c >= NBUF)
                def _(c=c, buf=buf):
                    drain_out(c - NBUF, buf)

                compute(buf)
                issue_out(c, buf)
            return carry

        lax.fori_loop(0, n_chunks // NBUF, pair_body, 0)
        for tail in range(NBUF):
            c = n_chunks - NBUF + tail
            if c >= 0:
                drain_out(c, c % NBUF)

    return sc_kernel


def kernel(x, table, lora_A, lora_B):
    batch, hist = x.shape
    n_tokens = batch * hist
    xf = x.reshape(-1).astype(jnp.int32)
    tbl_flat = table.reshape(VOCAB // 2, 2 * EMBED_DIM)
    a_flat = lora_A.reshape(-1)
    b_t = (lora_B * LORA_SCALING).T.astype(jnp.float32)
    out = _make_kernel(n_tokens)(tbl_flat, a_flat, b_t, xf)
    return out.reshape(batch, hist, EMBED_DIM)


# speedup vs baseline: 1.0144x; 1.0144x over previous
import functools

import jax
import jax.numpy as jnp
from jax import lax
from jax.experimental import pallas as pl
from jax.experimental.pallas import tpu as pltpu
from jax.experimental.pallas import tpu_sc as plsc

VOCAB = 1000000
EMBED_DIM = 64
LORA_R = 16
LORA_SCALING = 2.0
LANES = 16
CHUNK = 128
NBUF = 2


@functools.lru_cache(maxsize=None)
def _make_kernel(n_tokens: int):
    info = plsc.get_sparse_core_info()
    num_cores, num_subcores = info.num_cores, info.num_subcores
    num_workers = num_cores * num_subcores
    per_worker = n_tokens // num_workers
    assert per_worker * num_workers == n_tokens
    assert per_worker % CHUNK == 0
    n_chunks = per_worker // CHUNK
    d_groups = EMBED_DIM // LANES
    groups = CHUNK // LANES

    mesh = plsc.VectorSubcoreMesh(core_axis_name="c", subcore_axis_name="s")

    @functools.partial(
        pl.kernel,
        mesh=mesh,
        compiler_params=pltpu.CompilerParams(use_tc_tiling_on_sc=False),
        out_type=jax.ShapeDtypeStruct((n_tokens, EMBED_DIM), jnp.float32),
        scratch_types=[
            pltpu.VMEM((NBUF, CHUNK), jnp.int32),
            pltpu.VMEM((NBUF, CHUNK, EMBED_DIM), jnp.float32),
            pltpu.VMEM((NBUF, LORA_R, CHUNK), jnp.float32),
            pltpu.VMEM((NBUF, CHUNK, EMBED_DIM), jnp.float32),
            pltpu.VMEM((LORA_R, EMBED_DIM), jnp.float32),
            [pltpu.SemaphoreType.DMA] * NBUF,
            [pltpu.SemaphoreType.DMA] * NBUF,
            [pltpu.SemaphoreType.DMA] * NBUF,
        ],
    )
    def sc_kernel(tbl_hbm, a_hbm, bt_hbm, idx_hbm, out_hbm,
                  idx_v, rows_v, a_vt, out_v, b_v,
                  sem_rows, sem_a, sem_out):
        wid = lax.axis_index("s") * num_cores + lax.axis_index("c")
        base0 = wid * per_worker

        pltpu.sync_copy(bt_hbm, b_v)

        def fetch(c, buf):
            base = base0 + c * CHUNK
            pltpu.sync_copy(idx_hbm.at[pl.ds(base, CHUNK)], idx_v.at[buf])
            pltpu.async_copy(tbl_hbm.at[idx_v.at[buf]], rows_v.at[buf],
                             sem_rows[buf])
            for j in range(LORA_R):
                pltpu.async_copy(a_hbm.at[j].at[idx_v.at[buf]],
                                 a_vt.at[buf, j], sem_a[buf])

        def drain_gathers(buf):
            pltpu.make_async_copy(tbl_hbm.at[idx_v.at[buf]], rows_v.at[buf],
                                  sem_rows[buf]).wait()
            for j in range(LORA_R):
                pltpu.make_async_copy(a_hbm.at[j].at[idx_v.at[buf]],
                                      a_vt.at[buf, j], sem_a[buf]).wait()

        def compute(buf):
            def group_body(g, carry):
                t0 = g * LANES
                avs = [a_vt[buf, j, pl.ds(t0, LANES)] for j in range(LORA_R)]
                for half in range(2):
                    js = range(half * 8, half * 8 + 8)
                    bh = {j: [b_v[j, pl.ds(dg * LANES, LANES)]
                              for dg in range(d_groups)] for j in js}
                    src = rows_v if half == 0 else out_v
                    for l in range(LANES):
                        t = t0 + l
                        accs = [src[buf, t, pl.ds(dg * LANES, LANES)]
                                for dg in range(d_groups)]
                        for j in js:
                            s = avs[j][l]
                            for dg in range(d_groups):
                                accs[dg] = accs[dg] + s * bh[j][dg]
                        for dg in range(d_groups):
                            out_v[buf, t, pl.ds(dg * LANES, LANES)] = accs[dg]
                return carry

            lax.fori_loop(0, groups, group_body, 0)

        def issue_out(c, buf):
            base = base0 + c * CHUNK
            pltpu.async_copy(out_v.at[buf], out_hbm.at[pl.ds(base, CHUNK)],
                             sem_out[buf])

        def drain_out(c, buf):
            base = base0 + c * CHUNK
            pltpu.make_async_copy(out_v.at[buf],
                                  out_hbm.at[pl.ds(base, CHUNK)],
                                  sem_out[buf]).wait()

        fetch(0, 0)

        assert n_chunks % NBUF == 0

        def pair_body(c2, carry):
            for buf in range(NBUF):
                c = c2 * NBUF + buf
                nb = (buf + 1) % NBUF

                @pl.when(c + 1 < n_chunks)
                def _(c=c, nb=nb):
                    fetch(c + 1, nb)

                drain_gathers(buf)

                @pl.when(c >= NBUF)
                def _(c=c, buf=buf):
                    drain_out(c - NBUF, buf)

                compute(buf)
                issue_out(c, buf)
            return carry

        lax.fori_loop(0, n_chunks // NBUF, pair_body, 0)
        for tail in range(NBUF):
            c = n_chunks - NBUF + tail
            if c >= 0:
                drain_out(c, c % NBUF)

    return sc_kernel


def kernel(x, table, lora_A, lora_B):
    batch, hist = x.shape
    n_tokens = batch * hist
    xf = x.reshape(-1).astype(jnp.int32)
    b_t = (lora_B * LORA_SCALING).T.astype(jnp.float32)
    out = _make_kernel(n_tokens)(table, lora_A, b_t, xf)
    return out.reshape(batch, hist, EMBED_DIM)


# speedup vs baseline: 1.0535x; 1.0385x over previous
import functools

import jax
import jax.numpy as jnp
from jax import lax
from jax.experimental import pallas as pl
from jax.experimental.pallas import tpu as pltpu
from jax.experimental.pallas import tpu_sc as plsc

VOCAB = 1000000
A_COLS = 1000064
EMBED_DIM = 64
LORA_R = 16
LORA_SCALING = 2.0
LANES = 16
CHUNK = 128
NBUF = 2


@functools.lru_cache(maxsize=None)
def _make_kernel(n_tokens: int):
    info = plsc.get_sparse_core_info()
    num_cores, num_subcores = info.num_cores, info.num_subcores
    num_workers = num_cores * num_subcores
    per_worker = n_tokens // num_workers
    assert per_worker * num_workers == n_tokens
    assert per_worker % CHUNK == 0
    n_chunks = per_worker // CHUNK
    d_groups = EMBED_DIM // LANES
    groups = CHUNK // LANES

    mesh = plsc.VectorSubcoreMesh(core_axis_name="c", subcore_axis_name="s")

    @functools.partial(
        pl.kernel,
        mesh=mesh,
        compiler_params=pltpu.CompilerParams(use_tc_tiling_on_sc=False),
        out_type=jax.ShapeDtypeStruct((n_tokens, EMBED_DIM), jnp.float32),
        scratch_types=[
            pltpu.VMEM((NBUF, CHUNK), jnp.int32),
            pltpu.VMEM((NBUF, CHUNK, EMBED_DIM), jnp.float32),
            pltpu.VMEM((NBUF, LORA_R, CHUNK), jnp.float32),
            pltpu.VMEM((NBUF, CHUNK, EMBED_DIM), jnp.float32),
            pltpu.VMEM((LORA_R, EMBED_DIM), jnp.float32),
            [pltpu.SemaphoreType.DMA] * NBUF,
            [pltpu.SemaphoreType.DMA] * NBUF,
            [pltpu.SemaphoreType.DMA] * NBUF,
        ],
    )
    def sc_kernel(tbl_hbm, a_hbm, bt_hbm, idx_hbm, out_hbm,
                  idx_v, rows_v, a_vt, out_v, b_v,
                  sem_rows, sem_a, sem_out):
        wid = lax.axis_index("s") * num_cores + lax.axis_index("c")
        base0 = wid * per_worker

        pltpu.sync_copy(bt_hbm, b_v)

        def fetch(c, buf):
            base = base0 + c * CHUNK
            pltpu.sync_copy(idx_hbm.at[pl.ds(base, CHUNK)], idx_v.at[buf])
            pltpu.async_copy(tbl_hbm.at[idx_v.at[buf]], rows_v.at[buf],
                             sem_rows[buf])
            for j in range(LORA_R):
                pltpu.async_copy(a_hbm.at[j].at[idx_v.at[buf]],
                                 a_vt.at[buf, j], sem_a[buf])

        def drain_gathers(buf):
            pltpu.make_async_copy(tbl_hbm.at[idx_v.at[buf]], rows_v.at[buf],
                                  sem_rows[buf]).wait()
            for j in range(LORA_R):
                pltpu.make_async_copy(a_hbm.at[j].at[idx_v.at[buf]],
                                      a_vt.at[buf, j], sem_a[buf]).wait()

        def compute(buf):
            def group_body(g, carry):
                t0 = g * LANES
                avs = [a_vt[buf, j, pl.ds(t0, LANES)] for j in range(LORA_R)]
                for half in range(2):
                    js = range(half * 8, half * 8 + 8)
                    bh = {j: [b_v[j, pl.ds(dg * LANES, LANES)]
                              for dg in range(d_groups)] for j in js}
                    src = rows_v if half == 0 else out_v
                    for l in range(LANES):
                        t = t0 + l
                        accs = [src[buf, t, pl.ds(dg * LANES, LANES)]
                                for dg in range(d_groups)]
                        for j in js:
                            s = avs[j][l]
                            for dg in range(d_groups):
                                accs[dg] = accs[dg] + s * bh[j][dg]
                        for dg in range(d_groups):
                            out_v[buf, t, pl.ds(dg * LANES, LANES)] = accs[dg]
                return carry

            lax.fori_loop(0, groups, group_body, 0)

        def issue_out(c, buf):
            base = base0 + c * CHUNK
            pltpu.async_copy(out_v.at[buf], out_hbm.at[pl.ds(base, CHUNK)],
                             sem_out[buf])

        def drain_out(c, buf):
            base = base0 + c * CHUNK
            pltpu.make_async_copy(out_v.at[buf],
                                  out_hbm.at[pl.ds(base, CHUNK)],
                                  sem_out[buf]).wait()

        fetch(0, 0)

        assert n_chunks % NBUF == 0

        def pair_body(c2, carry):
            for buf in range(NBUF):
                c = c2 * NBUF + buf
                nb = (buf + 1) % NBUF

                @pl.when(c + 1 < n_chunks)
                def _(c=c, nb=nb):
                    fetch(c + 1, nb)

                drain_gathers(buf)

                @pl.when(c >= NBUF)
                def _(c=c, buf=buf):
                    drain_out(c - NBUF, buf)

                compute(buf)
                issue_out(c, buf)
            return carry

        lax.fori_loop(0, n_chunks // NBUF, pair_body, 0)
        for tail in range(NBUF):
            c = n_chunks - NBUF + tail
            if c >= 0:
                drain_out(c, c % NBUF)

    return sc_kernel


def kernel(x, table, lora_A, lora_B):
    batch, hist = x.shape
    n_tokens = batch * hist
    xf = x.reshape(-1).astype(jnp.int32)
    a_pad = jnp.pad(lora_A, ((0, 0), (0, A_COLS - VOCAB)))
    b_t = (lora_B * LORA_SCALING).T.astype(jnp.float32)
    out = _make_kernel(n_tokens)(table, a_pad, b_t, xf)
    return out.reshape(batch, hist, EMBED_DIM)


# speedup vs baseline: 1.2698x; 1.2053x over previous
import functools

import jax
import jax.numpy as jnp
from jax import lax
from jax.experimental import pallas as pl
from jax.experimental.pallas import tpu as pltpu
from jax.experimental.pallas import tpu_sc as plsc

VOCAB = 1000000
EMBED_DIM = 64
LORA_R = 16
LORA_SCALING = 2.0
LANES = 16
CHUNK = 128
NBUF = 2
PACK_BV = 2048
PACK_GRID = -(-VOCAB // PACK_BV)


def _pack_kernel(a_ref, o_ref):
    o_ref[...] = a_ref[...].T


def _pack_lora_a(lora_a):
    return pl.pallas_call(
        _pack_kernel,
        out_shape=jax.ShapeDtypeStruct((VOCAB, LORA_R), jnp.float32),
        grid=(PACK_GRID,),
        in_specs=[pl.BlockSpec((LORA_R, PACK_BV), lambda i: (0, i))],
        out_specs=pl.BlockSpec((PACK_BV, LORA_R), lambda i: (i, 0)),
    )(lora_a)


@functools.lru_cache(maxsize=None)
def _make_kernel(n_tokens: int):
    info = plsc.get_sparse_core_info()
    num_cores, num_subcores = info.num_cores, info.num_subcores
    num_workers = num_cores * num_subcores
    per_worker = n_tokens // num_workers
    assert per_worker * num_workers == n_tokens
    assert per_worker % CHUNK == 0
    n_chunks = per_worker // CHUNK
    d_groups = EMBED_DIM // LANES
    groups = CHUNK // LANES

    mesh = plsc.VectorSubcoreMesh(core_axis_name="c", subcore_axis_name="s")

    @functools.partial(
        pl.kernel,
        mesh=mesh,
        compiler_params=pltpu.CompilerParams(use_tc_tiling_on_sc=False),
        out_type=jax.ShapeDtypeStruct((n_tokens, EMBED_DIM), jnp.float32),
        scratch_types=[
            pltpu.VMEM((NBUF, CHUNK), jnp.int32),
            pltpu.VMEM((NBUF, CHUNK, EMBED_DIM), jnp.float32),
            pltpu.VMEM((NBUF, CHUNK, LORA_R), jnp.float32),
            pltpu.VMEM((NBUF, CHUNK, EMBED_DIM), jnp.float32),
            pltpu.VMEM((LORA_R, EMBED_DIM), jnp.float32),
            [pltpu.SemaphoreType.DMA] * NBUF,
            [pltpu.SemaphoreType.DMA] * NBUF,
            [pltpu.SemaphoreType.DMA] * NBUF,
        ],
    )
    def sc_kernel(tbl_hbm, ap_hbm, bt_hbm, idx_hbm, out_hbm,
                  idx_v, rows_v, a_vt, out_v, b_v,
                  sem_rows, sem_a, sem_out):
        wid = lax.axis_index("s") * num_cores + lax.axis_index("c")
        base0 = wid * per_worker

        pltpu.sync_copy(bt_hbm, b_v)

        def fetch(c, buf):
            base = base0 + c * CHUNK
            pltpu.sync_copy(idx_hbm.at[pl.ds(base, CHUNK)], idx_v.at[buf])
            pltpu.async_copy(tbl_hbm.at[idx_v.at[buf]], rows_v.at[buf],
                             sem_rows[buf])
            pltpu.async_copy(ap_hbm.at[idx_v.at[buf]], a_vt.at[buf],
                             sem_a[buf])

        def drain_gathers(buf):
            pltpu.make_async_copy(tbl_hbm.at[idx_v.at[buf]], rows_v.at[buf],
                                  sem_rows[buf]).wait()
            pltpu.make_async_copy(ap_hbm.at[idx_v.at[buf]], a_vt.at[buf],
                                  sem_a[buf]).wait()

        def compute(buf):
            def group_body(g, carry):
                t0 = g * LANES
                for half in range(2):
                    js = range(half * 8, half * 8 + 8)
                    bh = {j: [b_v[j, pl.ds(dg * LANES, LANES)]
                              for dg in range(d_groups)] for j in js}
                    src = rows_v if half == 0 else out_v
                    for l in range(LANES):
                        t = t0 + l
                        av = a_vt[buf, t, pl.ds(0, LORA_R)]
                        accs = [src[buf, t, pl.ds(dg * LANES, LANES)]
                                for dg in range(d_groups)]
                        for j in js:
                            s = av[j]
                            for dg in range(d_groups):
                                accs[dg] = accs[dg] + s * bh[j][dg]
                        for dg in range(d_groups):
                            out_v[buf, t, pl.ds(dg * LANES, LANES)] = accs[dg]
                return carry

            lax.fori_loop(0, groups, group_body, 0)

        def issue_out(c, buf):
            base = base0 + c * CHUNK
            pltpu.async_copy(out_v.at[buf], out_hbm.at[pl.ds(base, CHUNK)],
                             sem_out[buf])

        def drain_out(c, buf):
            base = base0 + c * CHUNK
            pltpu.make_async_copy(out_v.at[buf],
                                  out_hbm.at[pl.ds(base, CHUNK)],
                                  sem_out[buf]).wait()

        fetch(0, 0)

        assert n_chunks % NBUF == 0

        def pair_body(c2, carry):
            for buf in range(NBUF):
                c = c2 * NBUF + buf
                nb = (buf + 1) % NBUF

                @pl.when(c + 1 < n_chunks)
                def _(c=c, nb=nb):
                    fetch(c + 1, nb)

                drain_gathers(buf)

                @pl.when(c >= NBUF)
                def _(c=c, buf=buf):
                    drain_out(c - NBUF, buf)

                compute(buf)
                issue_out(c, buf)
            return carry

        lax.fori_loop(0, n_chunks // NBUF, pair_body, 0)
        for tail in range(NBUF):
            c = n_chunks - NBUF + tail
            if c >= 0:
                drain_out(c, c % NBUF)

    return sc_kernel


def kernel(x, table, lora_A, lora_B):
    batch, hist = x.shape
    n_tokens = batch * hist
    xf = x.reshape(-1).astype(jnp.int32)
    a_pack = _pack_lora_a(lora_A)
    b_t = (lora_B * LORA_SCALING).T.astype(jnp.float32)
    out = _make_kernel(n_tokens)(table, a_pack, b_t, xf)
    return out.reshape(batch, hist, EMBED_DIM)


# speedup vs baseline: 1.5493x; 1.2201x over previous
import functools

import jax
import jax.numpy as jnp
from jax import lax
from jax.experimental import pallas as pl
from jax.experimental.pallas import tpu as pltpu
from jax.experimental.pallas import tpu_sc as plsc

VOCAB = 1000000
EMBED_DIM = 64
LORA_R = 16
LORA_SCALING = 2.0
LANES = 16
CHUNK = 128
NBUF = 2
PACK_BV = 2048
PACK_GRID = -(-VOCAB // PACK_BV)


def _pack_kernel(a_ref, o_ref):
    t = a_ref[...].T
    o_ref[...] = jnp.pad(t, ((0, 0), (0, 128 - LORA_R)))


def _pack_lora_a(lora_a):
    return pl.pallas_call(
        _pack_kernel,
        out_shape=jax.ShapeDtypeStruct((VOCAB, 128), jnp.float32),
        grid=(PACK_GRID,),
        in_specs=[pl.BlockSpec((LORA_R, PACK_BV), lambda i: (0, i))],
        out_specs=pl.BlockSpec((PACK_BV, 128), lambda i: (i, 0)),
    )(lora_a)


@functools.lru_cache(maxsize=None)
def _make_kernel(n_tokens: int):
    info = plsc.get_sparse_core_info()
    num_cores, num_subcores = info.num_cores, info.num_subcores
    num_workers = num_cores * num_subcores
    per_worker = n_tokens // num_workers
    assert per_worker * num_workers == n_tokens
    assert per_worker % CHUNK == 0
    n_chunks = per_worker // CHUNK
    d_groups = EMBED_DIM // LANES
    groups = CHUNK // LANES

    mesh = plsc.VectorSubcoreMesh(core_axis_name="c", subcore_axis_name="s")

    @functools.partial(
        pl.kernel,
        mesh=mesh,
        compiler_params=pltpu.CompilerParams(use_tc_tiling_on_sc=False),
        out_type=jax.ShapeDtypeStruct((n_tokens, EMBED_DIM), jnp.float32),
        scratch_types=[
            pltpu.VMEM((NBUF, CHUNK), jnp.int32),
            pltpu.VMEM((NBUF, CHUNK, EMBED_DIM), jnp.float32),
            pltpu.VMEM((NBUF, CHUNK, 128), jnp.float32),
            pltpu.VMEM((NBUF, CHUNK, EMBED_DIM), jnp.float32),
            pltpu.VMEM((LORA_R, EMBED_DIM), jnp.float32),
            [pltpu.SemaphoreType.DMA] * NBUF,
            [pltpu.SemaphoreType.DMA] * NBUF,
            [pltpu.SemaphoreType.DMA] * NBUF,
        ],
    )
    def sc_kernel(tbl_hbm, ap_hbm, bt_hbm, idx_hbm, out_hbm,
                  idx_v, rows_v, a_vt, out_v, b_v,
                  sem_rows, sem_a, sem_out):
        wid = lax.axis_index("s") * num_cores + lax.axis_index("c")
        base0 = wid * per_worker

        pltpu.sync_copy(bt_hbm, b_v)

        def fetch(c, buf):
            base = base0 + c * CHUNK
            pltpu.sync_copy(idx_hbm.at[pl.ds(base, CHUNK)], idx_v.at[buf])
            pltpu.async_copy(tbl_hbm.at[idx_v.at[buf]], rows_v.at[buf],
                             sem_rows[buf])
            pltpu.async_copy(ap_hbm.at[idx_v.at[buf]], a_vt.at[buf],
                             sem_a[buf])

        def drain_gathers(buf):
            pltpu.make_async_copy(tbl_hbm.at[idx_v.at[buf]], rows_v.at[buf],
                                  sem_rows[buf]).wait()
            pltpu.make_async_copy(ap_hbm.at[idx_v.at[buf]], a_vt.at[buf],
                                  sem_a[buf]).wait()

        def compute(buf):
            def group_body(g, carry):
                t0 = g * LANES
                for half in range(2):
                    js = range(half * 8, half * 8 + 8)
                    bh = {j: [b_v[j, pl.ds(dg * LANES, LANES)]
                              for dg in range(d_groups)] for j in js}
                    src = rows_v if half == 0 else out_v
                    for l in range(LANES):
                        t = t0 + l
                        av = a_vt[buf, t, pl.ds(0, LORA_R)]
                        accs = [src[buf, t, pl.ds(dg * LANES, LANES)]
                                for dg in range(d_groups)]
                        for j in js:
                            s = av[j]
                            for dg in range(d_groups):
                                accs[dg] = accs[dg] + s * bh[j][dg]
                        for dg in range(d_groups):
                            out_v[buf, t, pl.ds(dg * LANES, LANES)] = accs[dg]
                return carry

            lax.fori_loop(0, groups, group_body, 0)

        def issue_out(c, buf):
            base = base0 + c * CHUNK
            pltpu.async_copy(out_v.at[buf], out_hbm.at[pl.ds(base, CHUNK)],
                             sem_out[buf])

        def drain_out(c, buf):
            base = base0 + c * CHUNK
            pltpu.make_async_copy(out_v.at[buf],
                                  out_hbm.at[pl.ds(base, CHUNK)],
                                  sem_out[buf]).wait()

        fetch(0, 0)

        assert n_chunks % NBUF == 0

        def pair_body(c2, carry):
            for buf in range(NBUF):
                c = c2 * NBUF + buf
                nb = (buf + 1) % NBUF

                @pl.when(c + 1 < n_chunks)
                def _(c=c, nb=nb):
                    fetch(c + 1, nb)

                drain_gathers(buf)

                @pl.when(c >= NBUF)
                def _(c=c, buf=buf):
                    drain_out(c - NBUF, buf)

                compute(buf)
                issue_out(c, buf)
            return carry

        lax.fori_loop(0, n_chunks // NBUF, pair_body, 0)
        for tail in range(NBUF):
            c = n_chunks - NBUF + tail
            if c >= 0:
                drain_out(c, c % NBUF)

    return sc_kernel


def kernel(x, table, lora_A, lora_B):
    batch, hist = x.shape
    n_tokens = batch * hist
    xf = x.reshape(-1).astype(jnp.int32)
    a_pack = _pack_lora_a(lora_A)
    b_t = (lora_B * LORA_SCALING).T.astype(jnp.float32)
    out = _make_kernel(n_tokens)(table, a_pack, b_t, xf)
    return out.reshape(batch, hist, EMBED_DIM)
